# Initial kernel scaffold; baseline (speedup 1.0000x reference)
import jax
import jax.numpy as jnp
from jax.experimental import pallas as pl


def kernel(X, Wf, bf, Wg, bg):
    raise NotImplementedError("write your pallas kernel here")


# speedup vs baseline: 64.0509x; 64.0509x over previous
import jax
import jax.numpy as jnp
from jax.experimental import pallas as pl
from jax.experimental.pallas import tpu as pltpu

N = 4096
D = 1024
BLK = 512
NBLK = N // BLK


def _body(x_ref, wgt_ref, bg_ref, out_ref, v_ref):
    i = pl.program_id(0)
    vblk = jnp.dot(x_ref[...], wgt_ref[...], preferred_element_type=jnp.float32)
    v_ref[pl.ds(i * BLK, BLK), :] = vblk + bg_ref[...]

    @pl.when(i == NBLK - 1)
    def _():
        v = v_ref[...]
        v1 = jnp.roll(v, shift=-1, axis=0)
        v2 = jnp.roll(v, shift=-2, axis=0)
        out_ref[...] = v[:, 0:1] * v1 + v[:, 1:2] * v2


def kernel(X, Wf, bf, Wg, bg):
    del Wf, bf
    wgt = Wg.T
    bg2 = bg.reshape(1, 2)
    return pl.pallas_call(
        _body,
        grid=(NBLK,),
        in_specs=[
            pl.BlockSpec((BLK, D), lambda i: (i, 0)),
            pl.BlockSpec((D, 2), lambda i: (0, 0)),
            pl.BlockSpec((1, 2), lambda i: (0, 0)),
        ],
        out_specs=pl.BlockSpec((N, 2), lambda i: (0, 0)),
        out_shape=jax.ShapeDtypeStruct((N, 2), jnp.float32),
        scratch_shapes=[pltpu.VMEM((N, 2), jnp.float32)],
    )(X, wgt, bg2)
